# baseline (speedup 1.0000x reference)
import functools

import jax
import jax.numpy as jnp
from jax import lax
from jax.experimental import pallas as pl
from jax.experimental.pallas import tpu as pltpu
from jax.experimental.pallas import tpu_sc as plsc

NODE = 10000
REL = 1000
TRI = 160000
DIM = 128
NC = 2
NS = 16
L = 16
NW = NC * NS
AF = REL * REL
AFP = 1000064
ZROWF = 62464
RPS = 624
RTAIL = NODE - NS * RPS
NODEP = 10112

CH = 128
NCHUNK = TRI // CH

_MESH = plsc.VectorSubcoreMesh(core_axis_name="c", subcore_axis_name="s",
                               num_cores=NC, num_subcores=NS)
_CPARAMS = pltpu.CompilerParams(needs_layout_passes=False)


def _zero_1d(ref, n):
    def body(i, _):
        ref[pl.ds(i * L, L)] = jnp.zeros((L,), jnp.float32)
        return 0
    lax.fori_loop(0, n // L, body, 0)


def _zero_2d(ref, rows):
    def body(i, _):
        for k in range(DIM // L):
            ref[i, pl.ds(k * L, L)] = jnp.zeros((L,), jnp.float32)
        return 0
    lax.fori_loop(0, rows, body, 0)


def _rsqrt(x):
    i = plsc.bitcast(x, jnp.int32)
    i = jnp.int32(0x5F3759DF) - jnp.right_shift(i, jnp.int32(1))
    y = plsc.bitcast(i, jnp.float32)
    for _ in range(3):
        y = y * (jnp.float32(1.5) - jnp.float32(0.5) * x * y * y)
    return y


def _edge_pass_a(s_hbm, d_hbm, fc_hbm, tf_hbm,
                 fs_out, fd_out, sumf_out, cnt_out,
                 accum, cntsh,
                 sbufs, dbufs, onesb, zerof, r0, r1, r2,
                 isem, gsem, wsem, ssem, csem):
    core = lax.axis_index("c")
    sid = lax.axis_index("s")
    wid = core * NS + sid

    _zero_2d(r0, CH)
    _zero_1d(zerof, CH)
    for g in range(CH // L):
        onesb[pl.ds(g * L, L)] = jnp.full((L,), 1.0, jnp.float32)
    base_r = sid * RPS
    for t in range(4):
        pltpu.sync_copy(r0, accum.at[pl.ds(base_r + t * CH, CH), :])
    pltpu.sync_copy(r0.at[pl.ds(0, RPS - 4 * CH)],
                    accum.at[pl.ds(base_r + 4 * CH, RPS - 4 * CH), :])

    @pl.when(sid == 0)
    def _():
        pltpu.sync_copy(r0.at[pl.ds(0, RTAIL)],
                        accum.at[pl.ds(NS * RPS, RTAIL), :])

    for j in range(5):
        t = sid + NS * j

        @pl.when(t < NODEP // CH)
        def _():
            pltpu.sync_copy(zerof, cntsh.at[pl.ds(t * CH, CH)])

    plsc.subcore_barrier()

    n_my = NCHUNK // NW + jnp.where(wid < NCHUNK % NW, 1, 0)

    def process(k, b):
        sb = sbufs[b]
        db = dbufs[b]
        pb = sbufs[1 - b]

        @pl.when(k >= 1)
        def _():
            pltpu.make_async_copy(r0, fs_out.at[pl.ds(0, CH), :], wsem).wait()
            pltpu.make_async_copy(r1, fd_out.at[pl.ds(0, CH), :], wsem).wait()
            pltpu.make_async_copy(r2, accum.at[pb], ssem).wait()
            pltpu.make_async_copy(onesb, cntsh.at[pb], csem).wait()

        base = (k * NW + wid) * CH
        pltpu.async_copy(s_hbm.at[pl.ds(base, CH)], sb, isem)
        pltpu.async_copy(d_hbm.at[pl.ds(base, CH)], db, isem)
        pltpu.make_async_copy(s_hbm.at[pl.ds(base, CH)], sb, isem).wait()
        pltpu.make_async_copy(d_hbm.at[pl.ds(base, CH)], db, isem).wait()

        pltpu.async_copy(fc_hbm.at[sb], r0, gsem)
        pltpu.async_copy(fc_hbm.at[db], r1, gsem)
        pltpu.async_copy(tf_hbm.at[db], r2, gsem)
        for r in (r0, r1, r2):
            pltpu.make_async_copy(fc_hbm.at[sb], r, gsem).wait()

        pltpu.async_copy(r0, fs_out.at[pl.ds(base, CH), :], wsem)
        pltpu.async_copy(r1, fd_out.at[pl.ds(base, CH), :], wsem)
        pltpu.async_copy(r2, accum.at[sb], ssem, add=True)
        pltpu.async_copy(onesb, cntsh.at[sb], csem, add=True)

    def pair_body(kk, _):
        for b in range(2):
            k = kk * 2 + b

            @pl.when(k < n_my)
            def _():
                process(k, b)
        return 0

    lax.fori_loop(0, (NCHUNK // NW + 2) // 2, pair_body, 0)

    pltpu.make_async_copy(r0, fs_out.at[pl.ds(0, CH), :], wsem).wait()
    pltpu.make_async_copy(r1, fd_out.at[pl.ds(0, CH), :], wsem).wait()
    pltpu.make_async_copy(r2, accum.at[sbufs[0]], ssem).wait()
    pltpu.make_async_copy(onesb, cntsh.at[sbufs[0]], csem).wait()

    plsc.subcore_barrier()

    pltpu.sync_copy(accum.at[pl.ds(base_r, RPS), :],
                    sumf_out.at[core, pl.ds(base_r, RPS), :])

    @pl.when(sid == 0)
    def _():
        pltpu.sync_copy(accum.at[pl.ds(NS * RPS, RTAIL), :],
                        sumf_out.at[core, pl.ds(NS * RPS, RTAIL), :])
        pltpu.sync_copy(cntsh, cnt_out.at[core, :])


def _rel_pass(rf_hbm, rv_hbm, a_out,
              ahalf,
              rfbufs, rvbufs, zbuf,
              isem, asem):
    core = lax.axis_index("c")
    sid = lax.axis_index("s")

    @pl.when(core == 0)
    def _():
        _zero_1d(zbuf, 1024)
        zbase = sid * ZROWF
        for t in range(ZROWF // 1024):
            pltpu.sync_copy(zbuf, ahalf.at[pl.ds(zbase + t * 1024, 1024)])

        @pl.when(sid == 0)
        def _():
            for t in range(5):
                pltpu.sync_copy(zbuf.at[pl.ds(0, 128)],
                                ahalf.at[pl.ds(NS * ZROWF + t * 128, 128)])

    plsc.subcore_barrier()

    @pl.when(core == 0)
    def _():
        n_my = NCHUNK // NS + jnp.where(sid < NCHUNK % NS, 1, 0)

        def process(k, b):
            rfb = rfbufs[b]
            rvb = rvbufs[b]

            @pl.when(k >= 2)
            def _():
                pltpu.make_async_copy(rvb, ahalf.at[rfb], asem).wait()

            base = (k * NS + sid) * CH
            pltpu.async_copy(rf_hbm.at[pl.ds(base, CH)], rfb, isem)
            pltpu.async_copy(rv_hbm.at[pl.ds(base, CH)], rvb, isem)
            pltpu.make_async_copy(rf_hbm.at[pl.ds(base, CH)], rfb, isem).wait()
            pltpu.make_async_copy(rv_hbm.at[pl.ds(base, CH)], rvb, isem).wait()
            pltpu.async_copy(rvb, ahalf.at[rfb], asem, add=True)

        def pair_body(kk, _):
            for b in range(2):
                k = kk * 2 + b

                @pl.when(k < n_my)
                def _():
                    process(k, b)
            return 0

        lax.fori_loop(0, (NCHUNK // NS + 2) // 2, pair_body, 0)

        for b in range(2):
            pltpu.make_async_copy(rvbufs[b], ahalf.at[rfbufs[b]], asem).wait()

    plsc.subcore_barrier()

    @pl.when(core == 0)
    def _():
        zbase = sid * ZROWF
        pltpu.sync_copy(ahalf.at[pl.ds(zbase, ZROWF)],
                        a_out.at[pl.ds(zbase, ZROWF)])

        @pl.when(sid == 0)
        def _():
            pltpu.sync_copy(ahalf.at[pl.ds(NS * ZROWF, AFP - NS * ZROWF)],
                            a_out.at[pl.ds(NS * ZROWF, AFP - NS * ZROWF)])


def _edge_pass_b(s_hbm, d_hbm, u_hbm, w_hbm, tf_hbm, ak_hbm,
                 r2_out, d2_out,
                 r2acc, d2sh,
                 sbufs, dbufs, ubuf, wbuf, fbuf, e2vs, akbuf, zerof,
                 isem, gsem, ssem, esem):
    core = lax.axis_index("c")
    sid = lax.axis_index("s")
    wid = core * NS + sid

    _zero_2d(fbuf, CH)
    _zero_1d(zerof, CH)
    base_r = sid * RPS
    for t in range(RPS // CH):
        pltpu.sync_copy(fbuf, r2acc.at[pl.ds(base_r + t * CH, CH), :])
    remr = RPS - (RPS // CH) * CH
    pltpu.sync_copy(fbuf.at[pl.ds(0, remr)],
                    r2acc.at[pl.ds(base_r + (RPS // CH) * CH, remr), :])

    @pl.when(sid == 0)
    def _():
        pltpu.sync_copy(fbuf.at[pl.ds(0, RTAIL)],
                        r2acc.at[pl.ds(NS * RPS, RTAIL), :])

    for j in range(5):
        t = sid + NS * j

        @pl.when(t < NODEP // CH)
        def _():
            pltpu.sync_copy(zerof, d2sh.at[pl.ds(t * CH, CH)])

    pltpu.sync_copy(ak_hbm, akbuf)
    plsc.subcore_barrier()

    akv = [akbuf[pl.ds(k * L, L)] for k in range(DIM // L)]
    lanes = lax.iota(jnp.int32, L)
    lane0 = lanes == 0

    n_my = NCHUNK // NW + jnp.where(wid < NCHUNK % NW, 1, 0)

    def process(k, b):
        sb = sbufs[b]
        db = dbufs[b]
        pb = sbufs[1 - b]
        e2v = e2vs[b]

        @pl.when(k >= 1)
        def _():
            pltpu.make_async_copy(fbuf, r2acc.at[pb], ssem).wait()
            pltpu.make_async_copy(e2vs[1 - b], d2sh.at[pb], esem).wait()

        base = (k * NW + wid) * CH
        pltpu.async_copy(s_hbm.at[pl.ds(base, CH)], sb, isem)
        pltpu.async_copy(d_hbm.at[pl.ds(base, CH)], db, isem)
        pltpu.make_async_copy(s_hbm.at[pl.ds(base, CH)], sb, isem).wait()
        pltpu.make_async_copy(d_hbm.at[pl.ds(base, CH)], db, isem).wait()

        pltpu.async_copy(u_hbm.at[sb], ubuf, gsem)
        pltpu.async_copy(w_hbm.at[db], wbuf, gsem)
        pltpu.async_copy(tf_hbm.at[db], fbuf, gsem)
        for r in (ubuf, wbuf, fbuf):
            pltpu.make_async_copy(u_hbm.at[sb], r, gsem).wait()

        def edge_body(e, _):
            vs = []
            fs = []
            vv = jnp.zeros((L,), jnp.float32)
            fv = jnp.zeros((L,), jnp.float32)
            av = jnp.zeros((L,), jnp.float32)
            for kk in range(DIM // L):
                sl = pl.ds(kk * L, L)
                u = ubuf[e, sl]
                w = wbuf[e, sl]
                f = fbuf[e, sl]
                v = u + w
                vs.append(v)
                fs.append(f)
                vv = vv + v * v
                fv = fv + f * v
                av = av + akv[kk] * v
            ssv = jnp.full((L,), jnp.maximum(jnp.sum(vv), jnp.float32(1e-24)))
            fvv = jnp.full((L,), jnp.sum(fv))
            avv = jnp.full((L,), jnp.sum(av))
            rsv = _rsqrt(ssv)
            e2 = jnp.exp(avv * rsv)
            c2 = jnp.float32(2.0) * e2 * fvv / ssv
            for kk in range(DIM // L):
                fbuf[e, pl.ds(kk * L, L)] = e2 * fs[kk] - c2 * vs[kk]
            plsc.store_scatter(e2v, [jnp.full((L,), e, jnp.int32)], e2,
                               mask=lane0)
            return 0

        lax.fori_loop(0, CH, edge_body, 0)

        pltpu.async_copy(fbuf, r2acc.at[sb], ssem, add=True)
        pltpu.async_copy(e2v, d2sh.at[sb], esem, add=True)

    def pair_body(kk, _):
        for b in range(2):
            k = kk * 2 + b

            @pl.when(k < n_my)
            def _():
                process(k, b)
        return 0

    lax.fori_loop(0, (NCHUNK // NW + 2) // 2, pair_body, 0)

    pltpu.make_async_copy(fbuf, r2acc.at[sbufs[0]], ssem).wait()
    pltpu.make_async_copy(e2vs[0], d2sh.at[sbufs[0]], esem).wait()

    plsc.subcore_barrier()

    pltpu.sync_copy(r2acc.at[pl.ds(base_r, RPS), :],
                    r2_out.at[core, pl.ds(base_r, RPS), :])

    @pl.when(sid == 0)
    def _():
        pltpu.sync_copy(r2acc.at[pl.ds(NS * RPS, RTAIL), :],
                        r2_out.at[core, pl.ds(NS * RPS, RTAIL), :])
        pltpu.sync_copy(d2sh, d2_out.at[core, :])


@functools.partial(
    pl.kernel,
    out_type=(
        jax.ShapeDtypeStruct((TRI, DIM), jnp.float32),
        jax.ShapeDtypeStruct((TRI, DIM), jnp.float32),
        jax.ShapeDtypeStruct((NC, NODE, DIM), jnp.float32),
        jax.ShapeDtypeStruct((NC, NODEP), jnp.float32),
    ),
    mesh=_MESH,
    compiler_params=_CPARAMS,
    scratch_types=[
        pltpu.VMEM_SHARED((NODE, DIM), jnp.float32),
        pltpu.VMEM_SHARED((NODEP,), jnp.float32),
        [pltpu.VMEM((CH,), jnp.int32)] * 2,
        [pltpu.VMEM((CH,), jnp.int32)] * 2,
        pltpu.VMEM((CH,), jnp.float32),
        pltpu.VMEM((CH,), jnp.float32),
        pltpu.VMEM((CH, DIM), jnp.float32),
        pltpu.VMEM((CH, DIM), jnp.float32),
        pltpu.VMEM((CH, DIM), jnp.float32),
        pltpu.SemaphoreType.DMA,
        pltpu.SemaphoreType.DMA,
        pltpu.SemaphoreType.DMA,
        pltpu.SemaphoreType.DMA,
        pltpu.SemaphoreType.DMA,
    ],
)
def _kernel_a(*refs):
    _edge_pass_a(*refs)


@functools.partial(
    pl.kernel,
    out_type=jax.ShapeDtypeStruct((AFP,), jnp.float32),
    mesh=_MESH,
    compiler_params=_CPARAMS,
    scratch_types=[
        pltpu.VMEM_SHARED((AFP,), jnp.float32),
        [pltpu.VMEM((CH,), jnp.int32)] * 2,
        [pltpu.VMEM((CH,), jnp.float32)] * 2,
        pltpu.VMEM((1024,), jnp.float32),
        pltpu.SemaphoreType.DMA,
        pltpu.SemaphoreType.DMA,
    ],
)
def _kernel_c(*refs):
    _rel_pass(*refs)


@functools.partial(
    pl.kernel,
    out_type=(
        jax.ShapeDtypeStruct((NC, NODE, DIM), jnp.float32),
        jax.ShapeDtypeStruct((NC, NODEP), jnp.float32),
    ),
    mesh=_MESH,
    compiler_params=_CPARAMS,
    scratch_types=[
        pltpu.VMEM_SHARED((NODE, DIM), jnp.float32),
        pltpu.VMEM_SHARED((NODEP,), jnp.float32),
        [pltpu.VMEM((CH,), jnp.int32)] * 2,
        [pltpu.VMEM((CH,), jnp.int32)] * 2,
        pltpu.VMEM((CH, DIM), jnp.float32),
        pltpu.VMEM((CH, DIM), jnp.float32),
        pltpu.VMEM((CH, DIM), jnp.float32),
        [pltpu.VMEM((CH,), jnp.float32)] * 2,
        pltpu.VMEM((DIM,), jnp.float32),
        pltpu.VMEM((CH,), jnp.float32),
        pltpu.SemaphoreType.DMA,
        pltpu.SemaphoreType.DMA,
        pltpu.SemaphoreType.DMA,
        pltpu.SemaphoreType.DMA,
    ],
)
def _kernel_b(*refs):
    _edge_pass_b(*refs)


def kernel(features, rel_emb, adj, r_index, r_val, features_c, Fussion,
           attn_kernel, attn_kernel_ent, proxy, gate_w, gate_b):
    F = features_c.astype(jnp.float32)
    tf = jnp.tanh(features.astype(jnp.float32))
    s = adj[0].astype(jnp.int32)
    d = adj[1].astype(jnp.int32)
    rflat = r_index[0].astype(jnp.int32) * REL + r_index[1].astype(jnp.int32)
    rv = r_val.astype(jnp.float32)

    fs_rows, fd_rows, sumf_parts, cnt = _kernel_a(s, d, F, tf)
    a_flat = _kernel_c(rflat, rv)
    SumF = sumf_parts.sum(axis=0)
    cs = cnt[:, :NODE].sum(axis=0)
    A = a_flat[:AF].reshape(REL, REL)

    tri = A @ rel_emb
    tn = tri / jnp.maximum(jnp.linalg.norm(tri, axis=1, keepdims=True), 1e-12)
    e1 = jnp.exp((tn @ attn_kernel)[:, 0])

    concat_fea = jnp.concatenate([fs_rows, fd_rows], axis=-1)
    mean = concat_fea.mean(axis=0, keepdims=True)
    Xc = concat_fea - mean
    C = Xc.T @ Xc
    eigvals, eigvecs = jnp.linalg.eigh(C)
    idx = jnp.argsort(eigvals)[::-1][:DIM]
    V = eigvecs[:, idx]
    S = jnp.sqrt(jnp.clip(eigvals[idx], 0.0, None))
    wv = jnp.power(S + 1e-05, -0.5)
    U = F @ (V[:DIM] * wv[None, :])
    Wm = F @ (V[DIM:] * wv[None, :])

    r2_parts, d2_parts = _kernel_b(s, d, U, Wm, tf, attn_kernel_ent[:, 0])
    R2 = r2_parts.sum(axis=0)
    D2 = d2_parts[:, :NODE].sum(axis=0)

    fd0 = tf[d[:REL]]
    dt = jnp.sum(fd0 * tn, axis=1)
    nr = fd0 - 2.0 * dt[:, None] * tn
    corr = e1[:, None] * nr - fd0
    R1 = SumF.at[s[:REL]].add(corr)
    D1 = cs.at[s[:REL]].add(e1 - 1.0)

    agg = (R1 / jnp.maximum(D1, 1e-30)[:, None]
           + 0.1 * R2 / jnp.maximum(D2, 1e-30)[:, None])
    return jnp.concatenate([tf, jnp.tanh(agg)], axis=-1)
